# Initial kernel scaffold; baseline (speedup 1.0000x reference)
import jax
import jax.numpy as jnp
from jax.experimental import pallas as pl


def kernel(s_feats, q_points, s_points, neighbor_indices, kernel_points, kp_weights, kp_bias, gn_weight, gn_bias):
    raise NotImplementedError("write your pallas kernel here")


# speedup vs baseline: 1.0509x; 1.0509x over previous
import functools

import jax
import jax.numpy as jnp
from jax import lax
from jax.experimental import pallas as pl
from jax.experimental.pallas import tpu as pltpu
from jax.experimental.pallas import tpu_sc as plsc

N = 10000
K = 32
CIN = 128
COUT = 128
KP = 15
SIGMA = 0.6
NGROUPS = 16
NEG_SLOPE = 0.1
EPS = 1e-5

NW = 32
CHUNK = 128
NPAD = 10240
PER_W = NPAD * K // NW
NCHUNK = PER_W // CHUNK
L = 16

BN = 200
NB = N // BN


def _sc_gather_body(feats_hbm, px_hbm, py_hbm, pz_hbm, idx_hbm,
                    gout_hbm, pxout_hbm, pyout_hbm, pzout_hbm,
                    idx_v, rows_v, pxt, pyt, pzt, pxb, pyb, pzb, sem_f):
    c = lax.axis_index("c")
    s = lax.axis_index("s")
    wid = s * 2 + c
    base = wid * PER_W

    pltpu.sync_copy(px_hbm, pxt)
    pltpu.sync_copy(py_hbm, pyt)
    pltpu.sync_copy(pz_hbm, pzt)

    def body(i, carry):
        off = pl.multiple_of(base + i * CHUNK, CHUNK)
        pltpu.sync_copy(idx_hbm.at[pl.ds(off, CHUNK)], idx_v)
        cp_f = pltpu.async_copy(feats_hbm.at[idx_v], rows_v, sem_f)
        for j in range(CHUNK // L):
            iv = idx_v[pl.ds(j * L, L)]
            pxb[pl.ds(j * L, L)] = plsc.load_gather(pxt, [iv])
            pyb[pl.ds(j * L, L)] = plsc.load_gather(pyt, [iv])
            pzb[pl.ds(j * L, L)] = plsc.load_gather(pzt, [iv])
        cp_f.wait()
        pltpu.sync_copy(rows_v, gout_hbm.at[pl.ds(off, CHUNK)])
        pltpu.sync_copy(pxb, pxout_hbm.at[pl.ds(off, CHUNK)])
        pltpu.sync_copy(pyb, pyout_hbm.at[pl.ds(off, CHUNK)])
        pltpu.sync_copy(pzb, pzout_hbm.at[pl.ds(off, CHUNK)])
        return carry

    lax.fori_loop(0, NCHUNK, body, 0)


def _make_sc_gather():
    flat = jax.ShapeDtypeStruct((NPAD * K,), jnp.float32)
    return functools.partial(
        pl.kernel,
        mesh=plsc.VectorSubcoreMesh(core_axis_name="c", subcore_axis_name="s"),
        compiler_params=pltpu.CompilerParams(needs_layout_passes=False),
        out_type=[
            jax.ShapeDtypeStruct((NPAD * K, CIN), jnp.float32),
            flat, flat, flat,
        ],
        scratch_types=[
            pltpu.VMEM((CHUNK,), jnp.int32),
            pltpu.VMEM((CHUNK, CIN), jnp.float32),
            pltpu.VMEM((N,), jnp.float32),
            pltpu.VMEM((N,), jnp.float32),
            pltpu.VMEM((N,), jnp.float32),
            pltpu.VMEM((CHUNK,), jnp.float32),
            pltpu.VMEM((CHUNK,), jnp.float32),
            pltpu.VMEM((CHUNK,), jnp.float32),
            pltpu.SemaphoreType.DMA,
        ],
    )(_sc_gather_body)


def _conv_body(g_ref, px_ref, py_ref, pz_ref, q_ref, kp_ref, w_ref, b_ref,
               y_ref, part_ref):
    gb = g_ref[...]
    px = px_ref[...]
    py = py_ref[...]
    pz = pz_ref[...]
    qx = q_ref[:, 0:1]
    qy = q_ref[:, 1:2]
    qz = q_ref[:, 2:3]

    nf = jnp.sum(gb, axis=-1)
    nnum = jnp.sum((nf > 0.0).astype(jnp.float32), axis=-1)
    nnum = jnp.maximum(nnum, 1.0)

    cols = []
    for p in range(KP):
        dx = px - (qx + kp_ref[p, 0])
        dy = py - (qy + kp_ref[p, 1])
        dz = pz - (qz + kp_ref[p, 2])
        sq = dx * dx + dy * dy + dz * dz
        w = jnp.maximum(1.0 - jnp.sqrt(sq) * (1.0 / SIGMA), 0.0)
        cols.append(jnp.sum(w[:, :, None] * gb, axis=1))
    wf = jnp.concatenate(cols, axis=-1)

    y = lax.dot_general(wf, w_ref[...], (((1,), (0,)), ((), ())),
                        preferred_element_type=jnp.float32)
    y = y / nnum[:, None] + b_ref[...]
    y_ref[...] = y

    s1 = jnp.sum(y, axis=0)
    s2 = jnp.sum(y * y, axis=0)
    part_ref[...] = jnp.concatenate(
        [s1[None, None, :], s2[None, None, :],
         jnp.zeros((1, 6, COUT), jnp.float32)], axis=1)


def _norm_body(y_ref, part_ref, gnw_ref, gnb_ref, o_ref):
    part = part_ref[...]
    s1 = jnp.sum(part[:, 0, :], axis=0)
    s2 = jnp.sum(part[:, 1, :], axis=0)

    ci = lax.broadcasted_iota(jnp.int32, (COUT, COUT), 0) // (COUT // NGROUPS)
    cj = lax.broadcasted_iota(jnp.int32, (COUT, COUT), 1) // (COUT // NGROUPS)
    same = (ci == cj).astype(jnp.float32)

    cnt = float(N * (COUT // NGROUPS))
    m = lax.dot_general(s1[None, :], same, (((1,), (0,)), ((), ())),
                        preferred_element_type=jnp.float32) / cnt
    e2 = lax.dot_general(s2[None, :], same, (((1,), (0,)), ((), ())),
                         preferred_element_type=jnp.float32) / cnt
    inv = lax.rsqrt(e2 - m * m + EPS)

    y = y_ref[...]
    x = (y - m) * inv * gnw_ref[...] + gnb_ref[...]
    o_ref[...] = jnp.where(x >= 0.0, x, NEG_SLOPE * x)


def _tc_calls(g3, px3, py3, pz3, q8, kp, wb, kp_bias, gn_weight, gn_bias):
    y, part = pl.pallas_call(
        _conv_body,
        grid=(NB,),
        in_specs=[
            pl.BlockSpec((BN, K, CIN), lambda i: (i, 0, 0)),
            pl.BlockSpec((BN, K), lambda i: (i, 0)),
            pl.BlockSpec((BN, K), lambda i: (i, 0)),
            pl.BlockSpec((BN, K), lambda i: (i, 0)),
            pl.BlockSpec((BN, 8), lambda i: (i, 0)),
            pl.BlockSpec(memory_space=pltpu.SMEM),
            pl.BlockSpec((KP * CIN, COUT), lambda i: (0, 0)),
            pl.BlockSpec((1, COUT), lambda i: (0, 0)),
        ],
        out_specs=[
            pl.BlockSpec((BN, COUT), lambda i: (i, 0)),
            pl.BlockSpec((1, 8, COUT), lambda i: (i, 0, 0)),
        ],
        out_shape=[
            jax.ShapeDtypeStruct((N, COUT), jnp.float32),
            jax.ShapeDtypeStruct((NB, 8, COUT), jnp.float32),
        ],
    )(g3, px3, py3, pz3, q8, kp, wb, kp_bias.reshape(1, COUT))

    out = pl.pallas_call(
        _norm_body,
        grid=(NB,),
        in_specs=[
            pl.BlockSpec((BN, COUT), lambda i: (i, 0)),
            pl.BlockSpec((NB, 8, COUT), lambda i: (0, 0, 0)),
            pl.BlockSpec((1, COUT), lambda i: (0, 0)),
            pl.BlockSpec((1, COUT), lambda i: (0, 0)),
        ],
        out_specs=pl.BlockSpec((BN, COUT), lambda i: (i, 0)),
        out_shape=jax.ShapeDtypeStruct((N, COUT), jnp.float32),
    )(y, part, gn_weight.reshape(1, COUT), gn_bias.reshape(1, COUT))
    return out


def kernel(s_feats, q_points, s_points, neighbor_indices, kernel_points,
           kp_weights, kp_bias, gn_weight, gn_bias):
    idx = neighbor_indices.astype(jnp.int32)
    idx = jnp.pad(idx, ((0, NPAD - N), (0, 0))).reshape(-1)
    q8 = jnp.pad(q_points, ((0, 0), (0, 5)))
    wb = kp_weights.reshape(KP * CIN, COUT)
    spx = s_points[:, 0]
    spy = s_points[:, 1]
    spz = s_points[:, 2]

    gflat, pxf, pyf, pzf = _make_sc_gather()(s_feats, spx, spy, spz, idx)
    g3 = gflat.reshape(NPAD, K, CIN)
    px3 = pxf.reshape(NPAD, K)
    py3 = pyf.reshape(NPAD, K)
    pz3 = pzf.reshape(NPAD, K)

    return _tc_calls(g3, px3, py3, pz3, q8, kernel_points, wb, kp_bias,
                     gn_weight, gn_bias)


# speedup vs baseline: 1.1213x; 1.0670x over previous
import functools

import jax
import jax.numpy as jnp
from jax import lax
from jax.experimental import pallas as pl
from jax.experimental.pallas import tpu as pltpu
from jax.experimental.pallas import tpu_sc as plsc

N = 10000
K = 32
CIN = 128
COUT = 128
KP = 15
SIGMA = 0.6
NGROUPS = 16
NEG_SLOPE = 0.1
EPS = 1e-5

NW = 32
CHUNK = 128
NPAD = 10240
PER_W = NPAD * K // NW
NCHUNK = PER_W // CHUNK
L = 16

BN = 200
NB = N // BN


def _sc_gather_body(feats_hbm, px_hbm, py_hbm, pz_hbm, idx_hbm,
                    gout_hbm, cout_hbm,
                    idx_v, rows_a, rows_b, pxt, pyt, pzt, cbuf_a, cbuf_b,
                    sem_f, sem_w):
    c = lax.axis_index("c")
    s = lax.axis_index("s")
    wid = s * 2 + c
    base = wid * PER_W
    cbase = wid * NCHUNK

    pltpu.sync_copy(idx_hbm.at[pl.ds(base, PER_W)], idx_v)
    pltpu.sync_copy(px_hbm, pxt)
    pltpu.sync_copy(py_hbm, pyt)
    pltpu.sync_copy(pz_hbm, pzt)

    def coords(i, cbuf):
        for j in range(CHUNK // L):
            iv = idx_v[pl.ds(i * CHUNK + j * L, L)]
            cbuf[pl.ds(j * L, L)] = plsc.load_gather(pxt, [iv])
            cbuf[pl.ds(CHUNK + j * L, L)] = plsc.load_gather(pyt, [iv])
            cbuf[pl.ds(2 * CHUNK + j * L, L)] = plsc.load_gather(pzt, [iv])

    def body(jj, carry):
        i0 = 2 * jj
        i1 = 2 * jj + 1
        cp_a = pltpu.async_copy(
            feats_hbm.at[idx_v.at[pl.ds(i0 * CHUNK, CHUNK)]], rows_a, sem_f)
        cp_b = pltpu.async_copy(
            feats_hbm.at[idx_v.at[pl.ds(i1 * CHUNK, CHUNK)]], rows_b, sem_f)
        coords(i0, cbuf_a)
        coords(i1, cbuf_b)
        cp_a.wait()
        w_a = pltpu.async_copy(
            rows_a, gout_hbm.at[pl.ds(pl.multiple_of(base + i0 * CHUNK, CHUNK),
                                      CHUNK)], sem_w)
        cp_b.wait()
        w_b = pltpu.async_copy(
            rows_b, gout_hbm.at[pl.ds(pl.multiple_of(base + i1 * CHUNK, CHUNK),
                                      CHUNK)], sem_w)
        w_c = pltpu.async_copy(
            cbuf_a, cout_hbm.at[pl.ds(pl.multiple_of((cbase + i0) * 3 * CHUNK,
                                                     CHUNK), 3 * CHUNK)], sem_w)
        w_d = pltpu.async_copy(
            cbuf_b, cout_hbm.at[pl.ds(pl.multiple_of((cbase + i1) * 3 * CHUNK,
                                                     CHUNK), 3 * CHUNK)], sem_w)
        w_a.wait()
        w_b.wait()
        w_c.wait()
        w_d.wait()
        return carry

    lax.fori_loop(0, NCHUNK // 2, body, 0)


def _make_sc_gather():
    return functools.partial(
        pl.kernel,
        mesh=plsc.VectorSubcoreMesh(core_axis_name="c", subcore_axis_name="s"),
        compiler_params=pltpu.CompilerParams(needs_layout_passes=False),
        out_type=[
            jax.ShapeDtypeStruct((NPAD * K, CIN), jnp.float32),
            jax.ShapeDtypeStruct((NW * NCHUNK * 3 * CHUNK,), jnp.float32),
        ],
        scratch_types=[
            pltpu.VMEM((PER_W,), jnp.int32),
            pltpu.VMEM((CHUNK, CIN), jnp.float32),
            pltpu.VMEM((CHUNK, CIN), jnp.float32),
            pltpu.VMEM((N,), jnp.float32),
            pltpu.VMEM((N,), jnp.float32),
            pltpu.VMEM((N,), jnp.float32),
            pltpu.VMEM((3 * CHUNK,), jnp.float32),
            pltpu.VMEM((3 * CHUNK,), jnp.float32),
            pltpu.SemaphoreType.DMA,
            pltpu.SemaphoreType.DMA,
        ],
    )(_sc_gather_body)


def _conv_body(g_ref, px_ref, py_ref, pz_ref, q_ref, kp_ref, w_ref, b_ref,
               y_ref, part_ref):
    gb = g_ref[...]
    px = px_ref[...]
    py = py_ref[...]
    pz = pz_ref[...]
    qx = q_ref[:, 0:1]
    qy = q_ref[:, 1:2]
    qz = q_ref[:, 2:3]

    nf = jnp.sum(gb, axis=-1)
    nnum = jnp.sum((nf > 0.0).astype(jnp.float32), axis=-1)
    nnum = jnp.maximum(nnum, 1.0)

    cols = []
    for p in range(KP):
        dx = px - (qx + kp_ref[p, 0])
        dy = py - (qy + kp_ref[p, 1])
        dz = pz - (qz + kp_ref[p, 2])
        sq = dx * dx + dy * dy + dz * dz
        w = jnp.maximum(1.0 - jnp.sqrt(sq) * (1.0 / SIGMA), 0.0)
        cols.append(jnp.sum(w[:, :, None] * gb, axis=1))
    wf = jnp.concatenate(cols, axis=-1)

    y = lax.dot_general(wf, w_ref[...], (((1,), (0,)), ((), ())),
                        preferred_element_type=jnp.float32)
    y = y / nnum[:, None] + b_ref[...]
    y_ref[...] = y

    s1 = jnp.sum(y, axis=0)
    s2 = jnp.sum(y * y, axis=0)
    part_ref[...] = jnp.concatenate(
        [s1[None, None, :], s2[None, None, :],
         jnp.zeros((1, 6, COUT), jnp.float32)], axis=1)


def _norm_body(y_ref, part_ref, gnw_ref, gnb_ref, o_ref):
    part = part_ref[...]
    s1 = jnp.sum(part[:, 0, :], axis=0)
    s2 = jnp.sum(part[:, 1, :], axis=0)

    ci = lax.broadcasted_iota(jnp.int32, (COUT, COUT), 0) // (COUT // NGROUPS)
    cj = lax.broadcasted_iota(jnp.int32, (COUT, COUT), 1) // (COUT // NGROUPS)
    same = (ci == cj).astype(jnp.float32)

    cnt = float(N * (COUT // NGROUPS))
    m = lax.dot_general(s1[None, :], same, (((1,), (0,)), ((), ())),
                        preferred_element_type=jnp.float32) / cnt
    e2 = lax.dot_general(s2[None, :], same, (((1,), (0,)), ((), ())),
                         preferred_element_type=jnp.float32) / cnt
    inv = lax.rsqrt(e2 - m * m + EPS)

    y = y_ref[...]
    x = (y - m) * inv * gnw_ref[...] + gnb_ref[...]
    o_ref[...] = jnp.where(x >= 0.0, x, NEG_SLOPE * x)


def _tc_calls(g3, px3, py3, pz3, q8, kp, wb, kp_bias, gn_weight, gn_bias):
    y, part = pl.pallas_call(
        _conv_body,
        grid=(NB,),
        in_specs=[
            pl.BlockSpec((BN, K, CIN), lambda i: (i, 0, 0)),
            pl.BlockSpec((BN, K), lambda i: (i, 0)),
            pl.BlockSpec((BN, K), lambda i: (i, 0)),
            pl.BlockSpec((BN, K), lambda i: (i, 0)),
            pl.BlockSpec((BN, 8), lambda i: (i, 0)),
            pl.BlockSpec(memory_space=pltpu.SMEM),
            pl.BlockSpec((KP * CIN, COUT), lambda i: (0, 0)),
            pl.BlockSpec((1, COUT), lambda i: (0, 0)),
        ],
        out_specs=[
            pl.BlockSpec((BN, COUT), lambda i: (i, 0)),
            pl.BlockSpec((1, 8, COUT), lambda i: (i, 0, 0)),
        ],
        out_shape=[
            jax.ShapeDtypeStruct((N, COUT), jnp.float32),
            jax.ShapeDtypeStruct((NB, 8, COUT), jnp.float32),
        ],
    )(g3, px3, py3, pz3, q8, kp, wb, kp_bias.reshape(1, COUT))

    out = pl.pallas_call(
        _norm_body,
        grid=(NB,),
        in_specs=[
            pl.BlockSpec((BN, COUT), lambda i: (i, 0)),
            pl.BlockSpec((NB, 8, COUT), lambda i: (0, 0, 0)),
            pl.BlockSpec((1, COUT), lambda i: (0, 0)),
            pl.BlockSpec((1, COUT), lambda i: (0, 0)),
        ],
        out_specs=pl.BlockSpec((BN, COUT), lambda i: (i, 0)),
        out_shape=jax.ShapeDtypeStruct((N, COUT), jnp.float32),
    )(y, part, gn_weight.reshape(1, COUT), gn_bias.reshape(1, COUT))
    return out


def kernel(s_feats, q_points, s_points, neighbor_indices, kernel_points,
           kp_weights, kp_bias, gn_weight, gn_bias):
    idx = neighbor_indices.astype(jnp.int32)
    idx = jnp.pad(idx, ((0, NPAD - N), (0, 0))).reshape(-1)
    q8 = jnp.pad(q_points, ((0, 0), (0, 5)))
    wb = kp_weights.reshape(KP * CIN, COUT)
    spx = s_points[:, 0]
    spy = s_points[:, 1]
    spz = s_points[:, 2]

    gflat, cflat = _make_sc_gather()(s_feats, spx, spy, spz, idx)
    g3 = gflat.reshape(NPAD, K, CIN)
    cf = cflat.reshape(NW * NCHUNK, 3 * CHUNK)
    px3 = cf[:, 0:CHUNK].reshape(NPAD, K)
    py3 = cf[:, CHUNK:2 * CHUNK].reshape(NPAD, K)
    pz3 = cf[:, 2 * CHUNK:3 * CHUNK].reshape(NPAD, K)

    return _tc_calls(g3, px3, py3, pz3, q8, kernel_points, wb, kp_bias,
                     gn_weight, gn_bias)


# speedup vs baseline: 2.0930x; 1.8666x over previous
import functools

import jax
import jax.numpy as jnp
from jax import lax
from jax.experimental import pallas as pl
from jax.experimental.pallas import tpu as pltpu
from jax.experimental.pallas import tpu_sc as plsc

N = 10000
K = 32
CIN = 128
COUT = 128
KP = 15
SIGMA = 0.6
NGROUPS = 16
NEG_SLOPE = 0.1
EPS = 1e-5

NW = 32
CHUNK = 128
NPAD = 10240
PER_W = NPAD * K // NW
NCHUNK = PER_W // CHUNK
L = 16

BN = 200
NB = N // BN


def _sc_gather_body(feats_hbm, px_hbm, py_hbm, pz_hbm, idx_hbm,
                    gout_hbm, cout_hbm,
                    idx_v, rows_a, rows_b, pxt, pyt, pzt, cbuf_a, cbuf_b,
                    sem_f, sem_w):
    c = lax.axis_index("c")
    s = lax.axis_index("s")
    wid = s * 2 + c
    base = wid * PER_W
    cbase = wid * NCHUNK

    pltpu.sync_copy(idx_hbm.at[pl.ds(base, PER_W)], idx_v)
    pltpu.sync_copy(px_hbm, pxt)
    pltpu.sync_copy(py_hbm, pyt)
    pltpu.sync_copy(pz_hbm, pzt)

    def coords(i, cbuf):
        for j in range(CHUNK // L):
            iv = idx_v[pl.ds(i * CHUNK + j * L, L)]
            cbuf[pl.ds(j * L, L)] = plsc.load_gather(pxt, [iv])
            cbuf[pl.ds(CHUNK + j * L, L)] = plsc.load_gather(pyt, [iv])
            cbuf[pl.ds(2 * CHUNK + j * L, L)] = plsc.load_gather(pzt, [iv])

    def body(jj, carry):
        i0 = 2 * jj
        i1 = 2 * jj + 1
        cp_a = pltpu.async_copy(
            feats_hbm.at[idx_v.at[pl.ds(i0 * CHUNK, CHUNK)]], rows_a, sem_f)
        cp_b = pltpu.async_copy(
            feats_hbm.at[idx_v.at[pl.ds(i1 * CHUNK, CHUNK)]], rows_b, sem_f)
        coords(i0, cbuf_a)
        coords(i1, cbuf_b)
        cp_a.wait()
        w_a = pltpu.async_copy(
            rows_a, gout_hbm.at[pl.ds(pl.multiple_of(base + i0 * CHUNK, CHUNK),
                                      CHUNK)], sem_w)
        cp_b.wait()
        w_b = pltpu.async_copy(
            rows_b, gout_hbm.at[pl.ds(pl.multiple_of(base + i1 * CHUNK, CHUNK),
                                      CHUNK)], sem_w)
        w_c = pltpu.async_copy(
            cbuf_a, cout_hbm.at[pl.ds(pl.multiple_of((cbase + i0) * 3 * CHUNK,
                                                     CHUNK), 3 * CHUNK)], sem_w)
        w_d = pltpu.async_copy(
            cbuf_b, cout_hbm.at[pl.ds(pl.multiple_of((cbase + i1) * 3 * CHUNK,
                                                     CHUNK), 3 * CHUNK)], sem_w)
        w_a.wait()
        w_b.wait()
        w_c.wait()
        w_d.wait()
        return carry

    lax.fori_loop(0, NCHUNK // 2, body, 0)


def _make_sc_gather():
    return functools.partial(
        pl.kernel,
        mesh=plsc.VectorSubcoreMesh(core_axis_name="c", subcore_axis_name="s"),
        compiler_params=pltpu.CompilerParams(needs_layout_passes=False),
        out_type=[
            jax.ShapeDtypeStruct((NPAD * K, CIN), jnp.float32),
            jax.ShapeDtypeStruct((NW * NCHUNK * 3 * CHUNK,), jnp.float32),
        ],
        scratch_types=[
            pltpu.VMEM((PER_W,), jnp.int32),
            pltpu.VMEM((CHUNK, CIN), jnp.float32),
            pltpu.VMEM((CHUNK, CIN), jnp.float32),
            pltpu.VMEM((N,), jnp.float32),
            pltpu.VMEM((N,), jnp.float32),
            pltpu.VMEM((N,), jnp.float32),
            pltpu.VMEM((3 * CHUNK,), jnp.float32),
            pltpu.VMEM((3 * CHUNK,), jnp.float32),
            pltpu.SemaphoreType.DMA,
            pltpu.SemaphoreType.DMA,
        ],
    )(_sc_gather_body)


def _conv_body(g_ref, px_ref, py_ref, pz_ref, q_ref, kp_ref, w_ref, b_ref,
               y_ref, part_ref):
    gb = g_ref[...]
    px = px_ref[...]
    py = py_ref[...]
    pz = pz_ref[...]
    qx = q_ref[:, 0:1]
    qy = q_ref[:, 1:2]
    qz = q_ref[:, 2:3]

    nf = jnp.sum(gb, axis=-1)
    nnum = jnp.sum((nf > 0.0).astype(jnp.float32), axis=-1)
    nnum = jnp.maximum(nnum, 1.0)

    kx = kp_ref[0:KP, 0:1]
    ky = kp_ref[0:KP, 1:2]
    kz = kp_ref[0:KP, 2:3]
    dx = (px - qx)[:, None, :] - kx[None, :, :]
    dy = (py - qy)[:, None, :] - ky[None, :, :]
    dz = (pz - qz)[:, None, :] - kz[None, :, :]
    sq = dx * dx + dy * dy + dz * dz
    w3 = jnp.maximum(1.0 - jnp.sqrt(sq) * (1.0 / SIGMA), 0.0)
    wf3 = lax.dot_general(w3, gb, (((2,), (1,)), ((0,), (0,))),
                          preferred_element_type=jnp.float32)
    wf = wf3.reshape(BN, KP * CIN)

    y = lax.dot_general(wf, w_ref[...], (((1,), (0,)), ((), ())),
                        preferred_element_type=jnp.float32)
    y = y / nnum[:, None] + b_ref[...]
    y_ref[...] = y

    s1 = jnp.sum(y, axis=0)
    s2 = jnp.sum(y * y, axis=0)
    part_ref[...] = jnp.concatenate(
        [s1[None, None, :], s2[None, None, :],
         jnp.zeros((1, 6, COUT), jnp.float32)], axis=1)


def _norm_body(y_ref, part_ref, gnw_ref, gnb_ref, o_ref):
    part = part_ref[...]
    s1 = jnp.sum(part[:, 0, :], axis=0)
    s2 = jnp.sum(part[:, 1, :], axis=0)

    ci = lax.broadcasted_iota(jnp.int32, (COUT, COUT), 0) // (COUT // NGROUPS)
    cj = lax.broadcasted_iota(jnp.int32, (COUT, COUT), 1) // (COUT // NGROUPS)
    same = (ci == cj).astype(jnp.float32)

    cnt = float(N * (COUT // NGROUPS))
    m = lax.dot_general(s1[None, :], same, (((1,), (0,)), ((), ())),
                        preferred_element_type=jnp.float32) / cnt
    e2 = lax.dot_general(s2[None, :], same, (((1,), (0,)), ((), ())),
                         preferred_element_type=jnp.float32) / cnt
    inv = lax.rsqrt(e2 - m * m + EPS)

    y = y_ref[...]
    x = (y - m) * inv * gnw_ref[...] + gnb_ref[...]
    o_ref[...] = jnp.where(x >= 0.0, x, NEG_SLOPE * x)


def _tc_calls(g3, px3, py3, pz3, q8, kp, wb, kp_bias, gn_weight, gn_bias):
    y, part = pl.pallas_call(
        _conv_body,
        grid=(NB,),
        in_specs=[
            pl.BlockSpec((BN, K, CIN), lambda i: (i, 0, 0)),
            pl.BlockSpec((BN, K), lambda i: (i, 0)),
            pl.BlockSpec((BN, K), lambda i: (i, 0)),
            pl.BlockSpec((BN, K), lambda i: (i, 0)),
            pl.BlockSpec((BN, 8), lambda i: (i, 0)),
            pl.BlockSpec((16, 16), lambda i: (0, 0)),
            pl.BlockSpec((KP * CIN, COUT), lambda i: (0, 0)),
            pl.BlockSpec((1, COUT), lambda i: (0, 0)),
        ],
        out_specs=[
            pl.BlockSpec((BN, COUT), lambda i: (i, 0)),
            pl.BlockSpec((1, 8, COUT), lambda i: (i, 0, 0)),
        ],
        out_shape=[
            jax.ShapeDtypeStruct((N, COUT), jnp.float32),
            jax.ShapeDtypeStruct((NB, 8, COUT), jnp.float32),
        ],
    )(g3, px3, py3, pz3, q8, jnp.pad(kp, ((0, 16 - KP), (0, 13))),
      wb, kp_bias.reshape(1, COUT))

    out = pl.pallas_call(
        _norm_body,
        grid=(NB,),
        in_specs=[
            pl.BlockSpec((BN, COUT), lambda i: (i, 0)),
            pl.BlockSpec((NB, 8, COUT), lambda i: (0, 0, 0)),
            pl.BlockSpec((1, COUT), lambda i: (0, 0)),
            pl.BlockSpec((1, COUT), lambda i: (0, 0)),
        ],
        out_specs=pl.BlockSpec((BN, COUT), lambda i: (i, 0)),
        out_shape=jax.ShapeDtypeStruct((N, COUT), jnp.float32),
    )(y, part, gn_weight.reshape(1, COUT), gn_bias.reshape(1, COUT))
    return out


def kernel(s_feats, q_points, s_points, neighbor_indices, kernel_points,
           kp_weights, kp_bias, gn_weight, gn_bias):
    idx = neighbor_indices.astype(jnp.int32)
    idx = jnp.pad(idx, ((0, NPAD - N), (0, 0))).reshape(-1)
    q8 = jnp.pad(q_points, ((0, 0), (0, 5)))
    wb = kp_weights.reshape(KP * CIN, COUT)
    spx = s_points[:, 0]
    spy = s_points[:, 1]
    spz = s_points[:, 2]

    gflat, cflat = _make_sc_gather()(s_feats, spx, spy, spz, idx)
    g3 = gflat.reshape(NPAD, K, CIN)
    cf = cflat.reshape(NW * NCHUNK, 3 * CHUNK)
    px3 = cf[:, 0:CHUNK].reshape(NPAD, K)
    py3 = cf[:, CHUNK:2 * CHUNK].reshape(NPAD, K)
    pz3 = cf[:, 2 * CHUNK:3 * CHUNK].reshape(NPAD, K)

    return _tc_calls(g3, px3, py3, pz3, q8, kernel_points, wb, kp_bias,
                     gn_weight, gn_bias)


# speedup vs baseline: 2.1446x; 1.0246x over previous
import functools

import jax
import jax.numpy as jnp
from jax import lax
from jax.experimental import pallas as pl
from jax.experimental.pallas import tpu as pltpu
from jax.experimental.pallas import tpu_sc as plsc

N = 10000
K = 32
CIN = 128
COUT = 128
KP = 15
SIGMA = 0.6
NGROUPS = 16
NEG_SLOPE = 0.1
EPS = 1e-5

NW = 32
CHUNK = 128
NPAD = 10240
PER_W = NPAD * K // NW
NCHUNK = PER_W // CHUNK
L = 16

BN = 200
NB = N // BN


def _sc_gather_body(feats_hbm, px_hbm, py_hbm, pz_hbm, idx_hbm,
                    gout_hbm, cout_hbm,
                    idx_v, rows_a, rows_b, rows_c, rows_d,
                    pxt, pyt, pzt, cbuf_a, cbuf_b, cbuf_c, cbuf_d,
                    sem_f1, sem_f2, sem_w1, sem_w2):
    c = lax.axis_index("c")
    s = lax.axis_index("s")
    wid = s * 2 + c
    base = wid * PER_W
    cbase = wid * NCHUNK

    pltpu.sync_copy(idx_hbm.at[pl.ds(base, PER_W)], idx_v)
    pltpu.sync_copy(px_hbm, pxt)
    pltpu.sync_copy(py_hbm, pyt)
    pltpu.sync_copy(pz_hbm, pzt)

    def coords(i, cbuf):
        for j in range(CHUNK // L):
            iv = idx_v[pl.ds(i * CHUNK + j * L, L)]
            cbuf[pl.ds(j * L, L)] = plsc.load_gather(pxt, [iv])
            cbuf[pl.ds(CHUNK + j * L, L)] = plsc.load_gather(pyt, [iv])
            cbuf[pl.ds(2 * CHUNK + j * L, L)] = plsc.load_gather(pzt, [iv])

    def do_phase(i0, rows_x, rows_y, cbuf_x, cbuf_y, sem_g, sem_wr):
        i1 = i0 + 1
        gx = pltpu.async_copy(
            feats_hbm.at[idx_v.at[pl.ds(i0 * CHUNK, CHUNK)]], rows_x, sem_g)
        gy = pltpu.async_copy(
            feats_hbm.at[idx_v.at[pl.ds(i1 * CHUNK, CHUNK)]], rows_y, sem_g)
        coords(i0, cbuf_x)
        coords(i1, cbuf_y)
        gx.wait()
        wx = pltpu.async_copy(
            rows_x, gout_hbm.at[pl.ds(pl.multiple_of(base + i0 * CHUNK, CHUNK),
                                      CHUNK)], sem_wr)
        gy.wait()
        wy = pltpu.async_copy(
            rows_y, gout_hbm.at[pl.ds(pl.multiple_of(base + i1 * CHUNK, CHUNK),
                                      CHUNK)], sem_wr)
        wcx = pltpu.async_copy(
            cbuf_x, cout_hbm.at[pl.ds(pl.multiple_of((cbase + i0) * 3 * CHUNK,
                                                     CHUNK), 3 * CHUNK)],
            sem_wr)
        wcy = pltpu.async_copy(
            cbuf_y, cout_hbm.at[pl.ds(pl.multiple_of((cbase + i1) * 3 * CHUNK,
                                                     CHUNK), 3 * CHUNK)],
            sem_wr)
        return (wx, wy, wcx, wcy)

    def wait_all(hs):
        for h in hs:
            h.wait()

    def body(jj, carry):
        i0 = 8 * jj
        h1 = do_phase(i0, rows_a, rows_b, cbuf_a, cbuf_b, sem_f1, sem_w1)
        h2 = do_phase(i0 + 2, rows_c, rows_d, cbuf_c, cbuf_d, sem_f2, sem_w2)
        wait_all(h1)
        h3 = do_phase(i0 + 4, rows_a, rows_b, cbuf_a, cbuf_b, sem_f1, sem_w1)
        wait_all(h2)
        h4 = do_phase(i0 + 6, rows_c, rows_d, cbuf_c, cbuf_d, sem_f2, sem_w2)
        wait_all(h3)
        wait_all(h4)
        return carry

    lax.fori_loop(0, NCHUNK // 8, body, 0)


def _make_sc_gather():
    return functools.partial(
        pl.kernel,
        mesh=plsc.VectorSubcoreMesh(core_axis_name="c", subcore_axis_name="s"),
        compiler_params=pltpu.CompilerParams(needs_layout_passes=False),
        out_type=[
            jax.ShapeDtypeStruct((NPAD * K, CIN), jnp.float32),
            jax.ShapeDtypeStruct((NW * NCHUNK * 3 * CHUNK,), jnp.float32),
        ],
        scratch_types=[
            pltpu.VMEM((PER_W,), jnp.int32),
            pltpu.VMEM((CHUNK, CIN), jnp.float32),
            pltpu.VMEM((CHUNK, CIN), jnp.float32),
            pltpu.VMEM((CHUNK, CIN), jnp.float32),
            pltpu.VMEM((CHUNK, CIN), jnp.float32),
            pltpu.VMEM((N,), jnp.float32),
            pltpu.VMEM((N,), jnp.float32),
            pltpu.VMEM((N,), jnp.float32),
            pltpu.VMEM((3 * CHUNK,), jnp.float32),
            pltpu.VMEM((3 * CHUNK,), jnp.float32),
            pltpu.VMEM((3 * CHUNK,), jnp.float32),
            pltpu.VMEM((3 * CHUNK,), jnp.float32),
            pltpu.SemaphoreType.DMA,
            pltpu.SemaphoreType.DMA,
            pltpu.SemaphoreType.DMA,
            pltpu.SemaphoreType.DMA,
        ],
    )(_sc_gather_body)


def _conv_body(g_ref, px_ref, py_ref, pz_ref, q_ref, kp_ref, w_ref, b_ref,
               y_ref, part_ref):
    gb = g_ref[...]
    px = px_ref[...]
    py = py_ref[...]
    pz = pz_ref[...]
    qx = q_ref[:, 0:1]
    qy = q_ref[:, 1:2]
    qz = q_ref[:, 2:3]

    nf = jnp.sum(gb, axis=-1)
    nnum = jnp.sum((nf > 0.0).astype(jnp.float32), axis=-1)
    nnum = jnp.maximum(nnum, 1.0)

    kx = kp_ref[0:KP, 0:1]
    ky = kp_ref[0:KP, 1:2]
    kz = kp_ref[0:KP, 2:3]
    dx = (px - qx)[:, None, :] - kx[None, :, :]
    dy = (py - qy)[:, None, :] - ky[None, :, :]
    dz = (pz - qz)[:, None, :] - kz[None, :, :]
    sq = dx * dx + dy * dy + dz * dz
    w3 = jnp.maximum(1.0 - jnp.sqrt(sq) * (1.0 / SIGMA), 0.0)
    wf3 = lax.dot_general(w3, gb, (((2,), (1,)), ((0,), (0,))),
                          preferred_element_type=jnp.float32)
    wf = wf3.reshape(BN, KP * CIN)

    y = lax.dot_general(wf, w_ref[...], (((1,), (0,)), ((), ())),
                        preferred_element_type=jnp.float32)
    y = y / nnum[:, None] + b_ref[...]
    y_ref[...] = y

    s1 = jnp.sum(y, axis=0)
    s2 = jnp.sum(y * y, axis=0)
    part_ref[...] = jnp.concatenate(
        [s1[None, None, :], s2[None, None, :],
         jnp.zeros((1, 6, COUT), jnp.float32)], axis=1)


def _norm_body(y_ref, part_ref, gnw_ref, gnb_ref, o_ref):
    part = part_ref[...]
    s1 = jnp.sum(part[:, 0, :], axis=0)
    s2 = jnp.sum(part[:, 1, :], axis=0)

    ci = lax.broadcasted_iota(jnp.int32, (COUT, COUT), 0) // (COUT // NGROUPS)
    cj = lax.broadcasted_iota(jnp.int32, (COUT, COUT), 1) // (COUT // NGROUPS)
    same = (ci == cj).astype(jnp.float32)

    cnt = float(N * (COUT // NGROUPS))
    m = lax.dot_general(s1[None, :], same, (((1,), (0,)), ((), ())),
                        preferred_element_type=jnp.float32) / cnt
    e2 = lax.dot_general(s2[None, :], same, (((1,), (0,)), ((), ())),
                         preferred_element_type=jnp.float32) / cnt
    inv = lax.rsqrt(e2 - m * m + EPS)

    y = y_ref[...]
    x = (y - m) * inv * gnw_ref[...] + gnb_ref[...]
    o_ref[...] = jnp.where(x >= 0.0, x, NEG_SLOPE * x)


def _tc_calls(g3, px3, py3, pz3, q8, kp, wb, kp_bias, gn_weight, gn_bias):
    y, part = pl.pallas_call(
        _conv_body,
        grid=(NB,),
        in_specs=[
            pl.BlockSpec((BN, K, CIN), lambda i: (i, 0, 0)),
            pl.BlockSpec((BN, K), lambda i: (i, 0)),
            pl.BlockSpec((BN, K), lambda i: (i, 0)),
            pl.BlockSpec((BN, K), lambda i: (i, 0)),
            pl.BlockSpec((BN, 8), lambda i: (i, 0)),
            pl.BlockSpec((16, 16), lambda i: (0, 0)),
            pl.BlockSpec((KP * CIN, COUT), lambda i: (0, 0)),
            pl.BlockSpec((1, COUT), lambda i: (0, 0)),
        ],
        out_specs=[
            pl.BlockSpec((BN, COUT), lambda i: (i, 0)),
            pl.BlockSpec((1, 8, COUT), lambda i: (i, 0, 0)),
        ],
        out_shape=[
            jax.ShapeDtypeStruct((N, COUT), jnp.float32),
            jax.ShapeDtypeStruct((NB, 8, COUT), jnp.float32),
        ],
    )(g3, px3, py3, pz3, q8, jnp.pad(kp, ((0, 16 - KP), (0, 13))),
      wb, kp_bias.reshape(1, COUT))

    out = pl.pallas_call(
        _norm_body,
        grid=(NB,),
        in_specs=[
            pl.BlockSpec((BN, COUT), lambda i: (i, 0)),
            pl.BlockSpec((NB, 8, COUT), lambda i: (0, 0, 0)),
            pl.BlockSpec((1, COUT), lambda i: (0, 0)),
            pl.BlockSpec((1, COUT), lambda i: (0, 0)),
        ],
        out_specs=pl.BlockSpec((BN, COUT), lambda i: (i, 0)),
        out_shape=jax.ShapeDtypeStruct((N, COUT), jnp.float32),
    )(y, part, gn_weight.reshape(1, COUT), gn_bias.reshape(1, COUT))
    return out


def kernel(s_feats, q_points, s_points, neighbor_indices, kernel_points,
           kp_weights, kp_bias, gn_weight, gn_bias):
    idx = neighbor_indices.astype(jnp.int32)
    idx = jnp.pad(idx, ((0, NPAD - N), (0, 0))).reshape(-1)
    q8 = jnp.pad(q_points, ((0, 0), (0, 5)))
    wb = kp_weights.reshape(KP * CIN, COUT)
    spx = s_points[:, 0]
    spy = s_points[:, 1]
    spz = s_points[:, 2]

    gflat, cflat = _make_sc_gather()(s_feats, spx, spy, spz, idx)
    g3 = gflat.reshape(NPAD, K, CIN)
    cf = cflat.reshape(NW * NCHUNK, 3 * CHUNK)
    px3 = cf[:, 0:CHUNK].reshape(NPAD, K)
    py3 = cf[:, CHUNK:2 * CHUNK].reshape(NPAD, K)
    pz3 = cf[:, 2 * CHUNK:3 * CHUNK].reshape(NPAD, K)

    return _tc_calls(g3, px3, py3, pz3, q8, kernel_points, wb, kp_bias,
                     gn_weight, gn_bias)
